# Initial kernel scaffold; baseline (speedup 1.0000x reference)
import jax
import jax.numpy as jnp
from jax.experimental import pallas as pl


def kernel(scores, k):
    raise NotImplementedError("write your pallas kernel here")


# speedup vs baseline: 10.4723x; 10.4723x over previous
import functools

import jax
import jax.numpy as jnp
from jax import lax
from jax.experimental import pallas as pl
from jax.experimental.pallas import tpu as pltpu

_K = 128
_N = 32768
_ROWS_PER_BLOCK = 16


def _body(k_ref, x_ref, o_ref):
    x = x_ref[...]
    xc = jnp.clip(x, 0.0, 1.0)
    s = jnp.sum(xc, axis=-1, keepdims=True)
    xb = lax.bitcast_convert_type(xc, jnp.int32)

    def vstep(_, carry):
        lo, hi = carry
        mid = (lo + hi) >> 1
        cnt = jnp.sum((xb >= mid).astype(jnp.int32), axis=-1, keepdims=True)
        ge = cnt >= _K
        return jnp.where(ge, mid, lo), jnp.where(ge, hi, mid)

    r = x.shape[0]
    lo0 = jnp.zeros((r, 1), jnp.int32)
    hi0 = jnp.full((r, 1), 0x3F800001, jnp.int32)
    lo, _ = lax.fori_loop(0, 31, vstep, (lo0, hi0))
    t = lo

    eq = xb == t
    n_gt = jnp.sum((xb > t).astype(jnp.int32), axis=-1, keepdims=True)
    need = _K - n_gt
    idx = lax.broadcasted_iota(jnp.int32, x.shape, 1)

    def jstep(_, carry):
        jlo, jhi = carry
        mid = (jlo + jhi) >> 1
        c = jnp.sum((eq & (idx < mid)).astype(jnp.int32), axis=-1, keepdims=True)
        geq = c >= need
        return jnp.where(geq, jlo, mid), jnp.where(geq, mid, jhi)

    jlo0 = jnp.zeros((r, 1), jnp.int32)
    jhi0 = jnp.full((r, 1), _N, jnp.int32)
    _, jhi = lax.fori_loop(0, 16, jstep, (jlo0, jhi0))

    mask = (xb > t) | (eq & (idx < jhi))
    cond = s > k_ref[0, 0]
    o_ref[...] = jnp.where(cond, jnp.where(mask, xc, 0.0), xc)


def kernel(scores, k):
    rows = scores.shape[0]
    kf = jnp.asarray(k, jnp.float32).reshape(1, 1)
    grid = (rows // _ROWS_PER_BLOCK,)
    return pl.pallas_call(
        _body,
        grid=grid,
        in_specs=[
            pl.BlockSpec(memory_space=pltpu.SMEM),
            pl.BlockSpec((_ROWS_PER_BLOCK, _N), lambda i: (i, 0)),
        ],
        out_specs=pl.BlockSpec((_ROWS_PER_BLOCK, _N), lambda i: (i, 0)),
        out_shape=jax.ShapeDtypeStruct(scores.shape, scores.dtype),
    )(kf, scores)
